# baseline (speedup 1.0000x reference)
import functools

import jax
import jax.numpy as jnp
from jax import lax
from jax.experimental import pallas as pl
from jax.experimental.pallas import tpu as pltpu, tpu_sc as plsc

N_NODES = 10000
N_EDGES = 320000
IN_DIM = 128
NPAD = 10240
NROW = NPAD // 128

_MESH = plsc.VectorSubcoreMesh(core_axis_name="c", subcore_axis_name="s")
_NC = _MESH.num_cores
_NS = _MESH.num_subcores
_NW = _NC * _NS
_NPS = NPAD // _NS
_NBLK = N_EDGES // 128
_BUFB = 158
_SC_PARAMS = pltpu.CompilerParams(needs_layout_passes=False)


def _blk_start(w):
    return 78 * w + jnp.maximum(0, w - 28)


def _newton_rsqrt(a):
    i = plsc.bitcast(a, jnp.int32)
    y = plsc.bitcast(jnp.int32(0x5F3759DF) - (i >> 1), jnp.float32)
    half = a * 0.5
    for _ in range(3):
        y = y * (1.5 - half * y * y)
    return y


@functools.partial(
    pl.kernel,
    out_type=(
        jax.ShapeDtypeStruct((_NW, NPAD), jnp.float32),
        jax.ShapeDtypeStruct((NPAD,), jnp.float32),
        jax.ShapeDtypeStruct((NPAD,), jnp.float32),
    ),
    mesh=_MESH,
    scratch_types=[
        pltpu.VMEM((2, _BUFB * 128), jnp.int32),
        pltpu.VMEM((NPAD,), jnp.float32),
        pltpu.VMEM((NPAD,), jnp.float32),
        pltpu.VMEM((_NS, _NPS), jnp.float32),
        pltpu.VMEM((_NPS,), jnp.float32),
        pltpu.VMEM((_NPS,), jnp.float32),
        pltpu.VMEM((_NPS,), jnp.float32),
        pltpu.VMEM((_NPS,), jnp.float32),
        pltpu.VMEM((NPAD,), jnp.float32),
        pltpu.VMEM_SHARED((_NS, NPAD), jnp.float32),
        pltpu.VMEM_SHARED((NPAD,), jnp.float32),
        pltpu.SemaphoreType.DMA,
        pltpu.SemaphoreType.DMA,
        pltpu.SemaphoreType.DMA,
        pltpu.SemaphoreType.DMA,
    ],
    compiler_params=_SC_PARAMS,
)
def _sc_main(ei_hbm, z_hbm, zeros_hbm,
             accp_hbm, dis_hbm, sterm_hbm,
             ei_v, hist_v, acc_v, red_v, z_v, zdis_v, dis_v, sterm_v,
             ztab_v, hist_sh, zdis_sh, sem1, sem2, sem3, sem4):
    cid = lax.axis_index("c")
    sid = lax.axis_index("s")
    wid = sid * _NC + cid

    base_blk = _blk_start(2 * sid)
    cp_ei = pltpu.async_copy(
        ei_hbm.at[:, pl.ds(base_blk * 128, _BUFB * 128)], ei_v, sem1)
    cp_z = pltpu.async_copy(z_hbm.at[pl.ds(sid * _NPS, _NPS)], z_v, sem2)
    cp_hz = pltpu.async_copy(zeros_hbm, hist_v, sem3)
    cp_az = pltpu.async_copy(zeros_hbm, acc_v, sem4)

    n_hist = 156 + 2 * (sid >= 14).astype(jnp.int32)
    cp_ei.wait()
    cp_hz.wait()
    ones = jnp.ones((16,), jnp.float32)

    @plsc.parallel_loop(0, n_hist * 8, 1, unroll=4)
    def hbody(i):
        d = ei_v[1, pl.ds(i * 16, 16)]
        plsc.addupdate_scatter(hist_v, [d], ones)

    pltpu.sync_copy(hist_v, hist_sh.at[sid])
    plsc.subcore_barrier()
    pltpu.sync_copy(hist_sh.at[:, pl.ds(sid * _NPS, _NPS)], red_v)
    cp_z.wait()

    @plsc.parallel_loop(0, _NPS // 16, 1, unroll=2)
    def rbody(v):
        o = v * 16
        s = red_v[0, pl.ds(o, 16)]
        for j in range(1, _NS):
            s = s + red_v[j, pl.ds(o, 16)]
        deg = s + 1.0
        dis = _newton_rsqrt(deg)
        z16 = z_v[pl.ds(o, 16)]
        dis_v[pl.ds(o, 16)] = dis
        zdis_v[pl.ds(o, 16)] = z16 * dis
        sterm_v[pl.ds(o, 16)] = z16 / deg

    pltpu.sync_copy(zdis_v, zdis_sh.at[pl.ds(sid * _NPS, _NPS)])

    @pl.when(cid == 0)
    def _():
        pltpu.sync_copy(dis_v, dis_hbm.at[pl.ds(sid * _NPS, _NPS)])
        pltpu.sync_copy(sterm_v, sterm_hbm.at[pl.ds(sid * _NPS, _NPS)])

    plsc.subcore_barrier()
    pltpu.sync_copy(zdis_sh, ztab_v)

    cp_az.wait()
    my_off = jnp.where(cid == 0, 0, (_blk_start(2 * sid + 1) - base_blk)) * 128
    n_acc = 78 + (wid >= 28).astype(jnp.int32)

    @plsc.parallel_loop(0, n_acc * 8, 1, unroll=4)
    def abody(i):
        s = ei_v[0, pl.ds(my_off + i * 16, 16)]
        vals = plsc.load_gather(ztab_v, [s])
        d = ei_v[1, pl.ds(my_off + i * 16, 16)]
        plsc.addupdate_scatter(acc_v, [d], vals)
    pltpu.sync_copy(acc_v, accp_hbm.at[wid])


def _tc_z_body(x_ref, w_ref, w2_ref, z_ref, wc_ref):
    r = pl.program_id(0)

    @pl.when(r == 0)
    def _():
        wc_ref[...] = w_ref[...] @ w2_ref[...]

    zb = jax.lax.dot_general(
        x_ref[...].reshape(8, 128, IN_DIM), wc_ref[...][:, 0],
        dimension_numbers=(((2,), (0,)), ((), ())),
    )
    z_ref[...] = zb


def _tc_final_body(accp_ref, dis_ref, sterm_ref, b_ref, w2_ref, b2_ref,
                   out_ref):
    acc = jnp.sum(accp_ref[...], axis=0)
    c = jnp.sum(b_ref[...] * w2_ref[...][:, 0]) + b2_ref[0]
    pre = dis_ref[...] * acc + sterm_ref[...] + c
    out_ref[...] = jax.nn.sigmoid(pre)


_tc_z = pl.pallas_call(
    _tc_z_body,
    grid=(NROW // 8,),
    in_specs=[
        pl.BlockSpec((1024, IN_DIM), lambda r: (r, 0)),
        pl.BlockSpec((IN_DIM, IN_DIM), lambda r: (0, 0)),
        pl.BlockSpec((IN_DIM, 1), lambda r: (0, 0)),
    ],
    out_specs=pl.BlockSpec((8, 128), lambda r: (r, 0)),
    out_shape=jax.ShapeDtypeStruct((NROW, 128), jnp.float32),
    scratch_shapes=[pltpu.VMEM((IN_DIM, 1), jnp.float32)],
)

_tc_final = pl.pallas_call(
    _tc_final_body,
    out_shape=jax.ShapeDtypeStruct((NPAD,), jnp.float32),
)


def kernel(x, edge_index, W, b, W2, b2):
    ei = edge_index.astype(jnp.int32)
    zeros = jnp.zeros((NPAD,), jnp.float32)

    z = _tc_z(x, W, W2)
    accp, dis, sterm = _sc_main(ei, z.reshape(NPAD), zeros)
    score = _tc_final(accp, dis, sterm, b, W2, b2)
    return score[:N_NODES]


# speedup vs baseline: 1.0889x; 1.0889x over previous
import functools

import jax
import jax.numpy as jnp
from jax import lax
from jax.experimental import pallas as pl
from jax.experimental.pallas import tpu as pltpu, tpu_sc as plsc

N_NODES = 10000
N_EDGES = 320000
IN_DIM = 128
NPAD = 10240
NROW = NPAD // 128

_MESH = plsc.VectorSubcoreMesh(core_axis_name="c", subcore_axis_name="s")
_NC = _MESH.num_cores
_NS = _MESH.num_subcores
_NW = _NC * _NS
_NPS = NPAD // _NS
_NBLK = N_EDGES // 128
_BUFB = 158
_SC_PARAMS = pltpu.CompilerParams(needs_layout_passes=False)


def _blk_start(w):
    return 78 * w + jnp.maximum(0, w - 28)


def _newton_rsqrt(a):
    i = plsc.bitcast(a, jnp.int32)
    y = plsc.bitcast(jnp.int32(0x5F3759DF) - (i >> 1), jnp.float32)
    half = a * 0.5
    for _ in range(3):
        y = y * (1.5 - half * y * y)
    return y


@functools.partial(
    pl.kernel,
    out_type=(
        jax.ShapeDtypeStruct((_NW, NPAD), jnp.float32),
        jax.ShapeDtypeStruct((NPAD,), jnp.float32),
        jax.ShapeDtypeStruct((NPAD,), jnp.float32),
    ),
    mesh=_MESH,
    scratch_types=[
        pltpu.VMEM((2, _BUFB * 128), jnp.int32),
        pltpu.VMEM((NPAD,), jnp.float32),
        pltpu.VMEM((NPAD,), jnp.float32),
        pltpu.VMEM((_NS, _NPS), jnp.float32),
        pltpu.VMEM((_NPS,), jnp.float32),
        pltpu.VMEM((_NPS,), jnp.float32),
        pltpu.VMEM((_NPS,), jnp.float32),
        pltpu.VMEM((_NPS,), jnp.float32),
        pltpu.VMEM((NPAD,), jnp.float32),
        pltpu.VMEM_SHARED((_NS, NPAD), jnp.float32),
        pltpu.VMEM_SHARED((NPAD,), jnp.float32),
        pltpu.SemaphoreType.DMA,
        pltpu.SemaphoreType.DMA,
        pltpu.SemaphoreType.DMA,
        pltpu.SemaphoreType.DMA,
    ],
    compiler_params=_SC_PARAMS,
)
def _sc_main(ei_hbm, z_hbm, zeros_hbm,
             accp_hbm, dis_hbm, sterm_hbm,
             ei_v, hist_v, acc_v, red_v, z_v, zdis_v, dis_v, sterm_v,
             ztab_v, hist_sh, zdis_sh, sem1, sem2, sem3, sem4):
    cid = lax.axis_index("c")
    sid = lax.axis_index("s")
    wid = sid * _NC + cid

    base_blk = _blk_start(2 * sid)
    cp_ei = pltpu.async_copy(
        ei_hbm.at[:, pl.ds(base_blk * 128, _BUFB * 128)], ei_v, sem1)
    cp_z = pltpu.async_copy(z_hbm.at[pl.ds(sid * _NPS, _NPS)], z_v, sem2)
    cp_hz = pltpu.async_copy(zeros_hbm, hist_v, sem3)
    cp_az = pltpu.async_copy(zeros_hbm, acc_v, sem4)

    n_hist = 156 + 2 * (sid >= 14).astype(jnp.int32)
    cp_ei.wait()
    cp_hz.wait()
    ones = jnp.ones((16,), jnp.float32)

    @plsc.parallel_loop(0, n_hist * 8, 1, unroll=4)
    def hbody(i):
        d = ei_v[1, pl.ds(i * 16, 16)]
        plsc.addupdate_scatter(hist_v, [d], ones)

    pltpu.sync_copy(hist_v, hist_sh.at[sid])
    plsc.subcore_barrier()
    pltpu.sync_copy(hist_sh.at[:, pl.ds(sid * _NPS, _NPS)], red_v)
    cp_z.wait()

    @plsc.parallel_loop(0, _NPS // 16, 1, unroll=2)
    def rbody(v):
        o = v * 16
        s = red_v[0, pl.ds(o, 16)]
        for j in range(1, _NS):
            s = s + red_v[j, pl.ds(o, 16)]
        deg = s + 1.0
        dis = _newton_rsqrt(deg)
        z16 = z_v[pl.ds(o, 16)]
        dis_v[pl.ds(o, 16)] = dis
        zdis_v[pl.ds(o, 16)] = z16 * dis
        sterm_v[pl.ds(o, 16)] = z16 / deg

    pltpu.sync_copy(zdis_v, zdis_sh.at[pl.ds(sid * _NPS, _NPS)])

    @pl.when(cid == 0)
    def _():
        pltpu.sync_copy(dis_v, dis_hbm.at[pl.ds(sid * _NPS, _NPS)])
        pltpu.sync_copy(sterm_v, sterm_hbm.at[pl.ds(sid * _NPS, _NPS)])

    plsc.subcore_barrier()
    pltpu.sync_copy(zdis_sh, ztab_v)

    cp_az.wait()
    my_off = jnp.where(cid == 0, 0, (_blk_start(2 * sid + 1) - base_blk)) * 128
    n_acc = 78 + (wid >= 28).astype(jnp.int32)

    @plsc.parallel_loop(0, n_acc * 8, 1, unroll=4)
    def abody(i):
        s = ei_v[0, pl.ds(my_off + i * 16, 16)]
        vals = plsc.load_gather(ztab_v, [s])
        d = ei_v[1, pl.ds(my_off + i * 16, 16)]
        plsc.addupdate_scatter(acc_v, [d], vals)
    pltpu.sync_copy(acc_v, accp_hbm.at[wid])


_NMAIN = (N_NODES // 128) * 128


def _tc_z_body(x_ref, w_ref, w2_ref, b_ref, b2_ref, z_ref, c_ref):
    wc = w_ref[...] @ w2_ref[...]
    xm = x_ref[0:_NMAIN, :].reshape(_NMAIN // 128, 128, IN_DIM)
    z_ref[0:_NMAIN // 128, :] = jax.lax.dot_general(
        xm, wc[:, 0],
        dimension_numbers=(((2,), (0,)), ((), ())),
    )
    xt = jnp.pad(
        x_ref[_NMAIN:N_NODES, :],
        ((0, (NROW - _NMAIN // 128) * 128 - (N_NODES - _NMAIN)), (0, 0)),
    ).reshape(NROW - _NMAIN // 128, 128, IN_DIM)
    z_ref[_NMAIN // 128:NROW, :] = jax.lax.dot_general(
        xt, wc[:, 0],
        dimension_numbers=(((2,), (0,)), ((), ())),
    )
    c = jnp.sum(b_ref[...] * w2_ref[...][:, 0]) + b2_ref[0]
    c_ref[...] = jnp.full((8, 128), c, jnp.float32)


def _tc_final_body(accp_ref, dis_ref, sterm_ref, c_ref, out_ref):
    acc = jnp.sum(accp_ref[...], axis=0)
    pre = dis_ref[...] * acc + sterm_ref[...] + c_ref[0, 0]
    out_ref[...] = jax.nn.sigmoid(pre)


_tc_z = pl.pallas_call(
    _tc_z_body,
    out_shape=[
        jax.ShapeDtypeStruct((NROW, 128), jnp.float32),
        jax.ShapeDtypeStruct((8, 128), jnp.float32),
    ],
)

_tc_final = pl.pallas_call(
    _tc_final_body,
    out_shape=jax.ShapeDtypeStruct((NPAD,), jnp.float32),
)


def kernel(x, edge_index, W, b, W2, b2):
    ei = edge_index.astype(jnp.int32)
    zeros = jnp.zeros((NPAD,), jnp.float32)

    z, cvec = _tc_z(x, W, W2, b, b2)
    accp, dis, sterm = _sc_main(ei, z.reshape(NPAD), zeros)
    score = _tc_final(accp, dis, sterm, cvec)
    return score[:N_NODES]
